# baseline (speedup 1.0000x reference)
import functools

import jax
import jax.numpy as jnp
from jax import lax
from jax.experimental import pallas as pl
from jax.experimental.pallas import tpu as pltpu
from jax.experimental.pallas import tpu_sc as plsc

_EMB = 64
_BATCH = 16384
_NC = 2
_NS = 16
_NW = _NC * _NS
_BPW = _BATCH // _NW
_CH = 128
_NCH = _BPW // _CH
_G = 16
_NG = _CH // _G


def _neg_sqrt(x):
    xc = jnp.maximum(x, jnp.float32(1e-30))
    i = plsc.bitcast(xc, jnp.int32)
    y = plsc.bitcast(jnp.int32(0x5F3759DF) - (i >> 1), jnp.float32)
    half = jnp.float32(0.5) * xc
    for _ in range(4):
        y = y * (jnp.float32(1.5) - half * y * y)
    return -(x * y)


def _sc_score(ent2, rel2, heads, tails, relations, neg_heads, neg_tails):
    mesh = plsc.VectorSubcoreMesh(core_axis_name="c", subcore_axis_name="s")
    out_t = jax.ShapeDtypeStruct((_BATCH,), jnp.float32)

    @functools.partial(
        pl.kernel,
        out_type=(out_t, out_t),
        mesh=mesh,
        scratch_types=(
            [pltpu.VMEM((_CH,), jnp.int32) for _ in range(5)]
            + [pltpu.VMEM((_CH,), jnp.int32) for _ in range(5)]
            + [pltpu.VMEM((_CH, 2 * _EMB), jnp.float32) for _ in range(5)]
            + [pltpu.VMEM((_G, _G), jnp.float32) for _ in range(2)]
            + [pltpu.VMEM((_CH,), jnp.float32) for _ in range(2)]
            + [pltpu.SemaphoreType.DMA]
        ),
        compiler_params=pltpu.CompilerParams(needs_layout_passes=False),
    )
    def k(ent_hbm, rel_hbm, h_hbm, t_hbm, r_hbm, nh_hbm, nt_hbm,
          pos_hbm, neg_hbm,
          hi_v, ti_v, ri_v, nhi_v, nti_v,
          hp_v, tp_v, rp_v, nhp_v, ntp_v,
          hr_v, tr_v, rr_v, nhr_v, ntr_v,
          sp_v, sn_v, pos_v, neg_v, sem):
        wid = lax.axis_index("s") * _NC + lax.axis_index("c")
        base = wid * _BPW
        ins = ((h_hbm, hi_v, hp_v, ent_hbm, hr_v),
               (t_hbm, ti_v, tp_v, ent_hbm, tr_v),
               (r_hbm, ri_v, rp_v, rel_hbm, rr_v),
               (nh_hbm, nhi_v, nhp_v, ent_hbm, nhr_v),
               (nt_hbm, nti_v, ntp_v, ent_hbm, ntr_v))
        iota = lax.iota(jnp.int32, _G)
        for c in range(_NCH):
            off = base + c * _CH
            for idx_hbm, idx_v, par_v, _, _ in ins:
                pltpu.sync_copy(idx_hbm.at[pl.ds(off, _CH)], idx_v)
            for _, idx_v, par_v, _, _ in ins:
                def halve(j, idx_v=idx_v, par_v=par_v):
                    sl = pl.ds(j * _G, _G)
                    e = idx_v[sl]
                    par_v[sl] = (e & 1) << 6
                    idx_v[sl] = e >> 1
                pl.loop(0, _CH // _G)(halve)
            copies = [pltpu.async_copy(tab_hbm.at[idx_v], rows_v, sem)
                      for _, idx_v, _, tab_hbm, rows_v in ins]
            for cp in copies:
                cp.wait()

            @pl.loop(0, _NG)
            def _(g):
                row0 = g * _G
                sl16 = pl.ds(row0, _G)
                ph16 = hp_v[sl16]
                pt16 = tp_v[sl16]
                pr16 = rp_v[sl16]
                pnh16 = nhp_v[sl16]
                pnt16 = ntp_v[sl16]
                for i in range(_G):
                    row = row0 + i
                    ph, pt, pr = ph16[i], pt16[i], pr16[i]
                    pnh, pnt = pnh16[i], pnt16[i]
                    p = jnp.zeros((_G,), jnp.float32)
                    pn = jnp.zeros((_G,), jnp.float32)
                    for kk in range(_EMB // _G):
                        o = kk * _G
                        rv = rr_v[row, pl.ds(pr + o, _G)]
                        d = hr_v[row, pl.ds(ph + o, _G)] + rv \
                            - tr_v[row, pl.ds(pt + o, _G)]
                        p = p + d * d
                        dn = nhr_v[row, pl.ds(pnh + o, _G)] + rv \
                            - ntr_v[row, pl.ds(pnt + o, _G)]
                        pn = pn + dn * dn
                    sp_v[i, :] = p
                    sn_v[i, :] = pn
                accp = jnp.zeros((_G,), jnp.float32)
                accn = jnp.zeros((_G,), jnp.float32)
                for j in range(_G):
                    col = jnp.full((_G,), j, jnp.int32)
                    accp = accp + plsc.load_gather(sp_v, [iota, col])
                    accn = accn + plsc.load_gather(sn_v, [iota, col])
                pos_v[pl.ds(row0, _G)] = _neg_sqrt(accp)
                neg_v[pl.ds(row0, _G)] = _neg_sqrt(accn)

            pltpu.sync_copy(pos_v, pos_hbm.at[pl.ds(off, _CH)])
            pltpu.sync_copy(neg_v, neg_hbm.at[pl.ds(off, _CH)])

    return k(ent2, rel2, heads, tails, relations, neg_heads, neg_tails)


def kernel(heads, tails, relations, negative_heads, negative_tails, ent_emb, rel_emb):
    idx = [x.astype(jnp.int32) for x in
           (heads, tails, relations, negative_heads, negative_tails)]
    ent2 = ent_emb.reshape(ent_emb.shape[0] // 2, 2 * _EMB)
    rel2 = rel_emb.reshape(rel_emb.shape[0] // 2, 2 * _EMB)
    pos, neg = _sc_score(ent2, rel2, *idx)
    return (pos, neg)


# speedup vs baseline: 1.6877x; 1.6877x over previous
import functools

import jax
import jax.numpy as jnp
from jax import lax
from jax.experimental import pallas as pl
from jax.experimental.pallas import tpu as pltpu
from jax.experimental.pallas import tpu_sc as plsc

_EMB = 64
_BATCH = 16384
_NC = 2
_NS = 16
_NW = _NC * _NS
_BPW = _BATCH // _NW
_CH = 128
_NCH = _BPW // _CH
_G = 16
_NG = _CH // _G


def _neg_sqrt(x):
    xc = jnp.maximum(x, jnp.float32(1e-30))
    i = plsc.bitcast(xc, jnp.int32)
    y = plsc.bitcast(jnp.int32(0x5F3759DF) - (i >> 1), jnp.float32)
    half = jnp.float32(0.5) * xc
    for _ in range(4):
        y = y * (jnp.float32(1.5) - half * y * y)
    return -(x * y)


def _sc_score(ent2, rel2, heads, tails, relations, neg_heads, neg_tails):
    mesh = plsc.VectorSubcoreMesh(core_axis_name="c", subcore_axis_name="s")
    out_t = jax.ShapeDtypeStruct((_BATCH,), jnp.float32)

    @functools.partial(
        pl.kernel,
        out_type=(out_t, out_t),
        mesh=mesh,
        scratch_types=(
            [pltpu.VMEM((_CH,), jnp.int32) for _ in range(5)]
            + [pltpu.VMEM((_CH,), jnp.int32) for _ in range(5)]
            + [pltpu.VMEM((_CH, 2 * _EMB), jnp.float32) for _ in range(5)]
            + [pltpu.VMEM((_G, _G), jnp.float32) for _ in range(2)]
            + [pltpu.VMEM((_CH,), jnp.float32) for _ in range(2)]
            + [pltpu.SemaphoreType.DMA]
        ),
        compiler_params=pltpu.CompilerParams(needs_layout_passes=False),
    )
    def k(ent_hbm, rel_hbm, h_hbm, t_hbm, r_hbm, nh_hbm, nt_hbm,
          pos_hbm, neg_hbm,
          hi_v, ti_v, ri_v, nhi_v, nti_v,
          hp_v, tp_v, rp_v, nhp_v, ntp_v,
          hr_v, tr_v, rr_v, nhr_v, ntr_v,
          sp_v, sn_v, pos_v, neg_v, sem):
        wid = lax.axis_index("s") * _NC + lax.axis_index("c")
        base = wid * _BPW
        ins = ((h_hbm, hi_v, hp_v, ent_hbm, hr_v),
               (t_hbm, ti_v, tp_v, ent_hbm, tr_v),
               (r_hbm, ri_v, rp_v, rel_hbm, rr_v),
               (nh_hbm, nhi_v, nhp_v, ent_hbm, nhr_v),
               (nt_hbm, nti_v, ntp_v, ent_hbm, ntr_v))
        iota = lax.iota(jnp.int32, _G)
        for c in range(_NCH):
            off = base + c * _CH
            for idx_hbm, idx_v, par_v, _, _ in ins:
                pltpu.sync_copy(idx_hbm.at[pl.ds(off, _CH)], idx_v)
            for _, idx_v, par_v, _, _ in ins:
                def halve(j, idx_v=idx_v, par_v=par_v):
                    sl = pl.ds(j * _G, _G)
                    e = idx_v[sl]
                    par_v[sl] = ((e >> 11) & 1) << 6
                    idx_v[sl] = ((e >> 12) << 11) + (e & 2047)
                pl.loop(0, _CH // _G)(halve)
            copies = [pltpu.async_copy(tab_hbm.at[idx_v], rows_v, sem)
                      for _, idx_v, _, tab_hbm, rows_v in ins]
            for cp in copies:
                cp.wait()

            @pl.loop(0, _NG)
            def _(g):
                row0 = g * _G
                sl16 = pl.ds(row0, _G)
                ph16 = hp_v[sl16]
                pt16 = tp_v[sl16]
                pr16 = rp_v[sl16]
                pnh16 = nhp_v[sl16]
                pnt16 = ntp_v[sl16]
                for i in range(_G):
                    row = row0 + i
                    ph, pt, pr = ph16[i], pt16[i], pr16[i]
                    pnh, pnt = pnh16[i], pnt16[i]
                    p = jnp.zeros((_G,), jnp.float32)
                    pn = jnp.zeros((_G,), jnp.float32)
                    for kk in range(_EMB // _G):
                        o = kk * _G
                        rv = rr_v[row, pl.ds(pr + o, _G)]
                        d = hr_v[row, pl.ds(ph + o, _G)] + rv \
                            - tr_v[row, pl.ds(pt + o, _G)]
                        p = p + d * d
                        dn = nhr_v[row, pl.ds(pnh + o, _G)] + rv \
                            - ntr_v[row, pl.ds(pnt + o, _G)]
                        pn = pn + dn * dn
                    sp_v[i, :] = p
                    sn_v[i, :] = pn
                accp = jnp.zeros((_G,), jnp.float32)
                accn = jnp.zeros((_G,), jnp.float32)
                for j in range(_G):
                    col = jnp.full((_G,), j, jnp.int32)
                    accp = accp + plsc.load_gather(sp_v, [iota, col])
                    accn = accn + plsc.load_gather(sn_v, [iota, col])
                pos_v[pl.ds(row0, _G)] = _neg_sqrt(accp)
                neg_v[pl.ds(row0, _G)] = _neg_sqrt(accn)

            pltpu.sync_copy(pos_v, pos_hbm.at[pl.ds(off, _CH)])
            pltpu.sync_copy(neg_v, neg_hbm.at[pl.ds(off, _CH)])

    return k(ent2, rel2, heads, tails, relations, neg_heads, neg_tails)


_TW = 4096


def _tc_pack(tab_t):
    n = tab_t.shape[1]
    nb = (n + _TW - 1) // _TW

    def body(x_ref, o_ref):
        y = jnp.transpose(x_ref[...])
        o_ref[...] = jnp.concatenate(
            [y[: _TW // 2], y[_TW // 2:]], axis=1)

    return pl.pallas_call(
        body,
        grid=(nb,),
        in_specs=[pl.BlockSpec((_EMB, _TW), lambda i: (0, i))],
        out_specs=pl.BlockSpec((_TW // 2, 2 * _EMB), lambda i: (i, 0)),
        out_shape=jax.ShapeDtypeStruct((nb * _TW // 2, 2 * _EMB), jnp.float32),
    )(tab_t)


def kernel(heads, tails, relations, negative_heads, negative_tails, ent_emb, rel_emb):
    idx = [x.astype(jnp.int32) for x in
           (heads, tails, relations, negative_heads, negative_tails)]
    ent2 = _tc_pack(jnp.swapaxes(ent_emb, 0, 1))
    rel2 = _tc_pack(jnp.swapaxes(rel_emb, 0, 1))
    pos, neg = _sc_score(ent2, rel2, *idx)
    return (pos, neg)


# speedup vs baseline: 1.6890x; 1.0008x over previous
import functools

import jax
import jax.numpy as jnp
from jax import lax
from jax.experimental import pallas as pl
from jax.experimental.pallas import tpu as pltpu
from jax.experimental.pallas import tpu_sc as plsc

_EMB = 64
_BATCH = 16384
_NC = 2
_NS = 16
_NW = _NC * _NS
_BPW = _BATCH // _NW
_CH = 128
_NCH = _BPW // _CH
_G = 16
_NG = _CH // _G


def _neg_sqrt(x):
    xc = jnp.maximum(x, jnp.float32(1e-30))
    i = plsc.bitcast(xc, jnp.int32)
    y = plsc.bitcast(jnp.int32(0x5F3759DF) - (i >> 1), jnp.float32)
    half = jnp.float32(0.5) * xc
    for _ in range(4):
        y = y * (jnp.float32(1.5) - half * y * y)
    return -(x * y)


def _sc_score(ent2, rel2, heads, tails, relations, neg_heads, neg_tails):
    mesh = plsc.VectorSubcoreMesh(core_axis_name="c", subcore_axis_name="s")
    out_t = jax.ShapeDtypeStruct((_BATCH,), jnp.float32)

    @functools.partial(
        pl.kernel,
        out_type=(out_t, out_t),
        mesh=mesh,
        scratch_types=(
            [pltpu.VMEM((_CH,), jnp.int32) for _ in range(5)]
            + [pltpu.VMEM((_CH,), jnp.int32) for _ in range(5)]
            + [pltpu.VMEM((_CH, 2 * _EMB), jnp.float32) for _ in range(5)]
            + [pltpu.VMEM((_G, _G), jnp.float32) for _ in range(2)]
            + [pltpu.VMEM((_CH,), jnp.float32) for _ in range(2)]
            + [pltpu.SemaphoreType.DMA]
        ),
        compiler_params=pltpu.CompilerParams(needs_layout_passes=False),
    )
    def k(ent_hbm, rel_hbm, h_hbm, t_hbm, r_hbm, nh_hbm, nt_hbm,
          pos_hbm, neg_hbm,
          hi_v, ti_v, ri_v, nhi_v, nti_v,
          hp_v, tp_v, rp_v, nhp_v, ntp_v,
          hr_v, tr_v, rr_v, nhr_v, ntr_v,
          sp_v, sn_v, pos_v, neg_v, sem):
        wid = lax.axis_index("s") * _NC + lax.axis_index("c")
        base = wid * _BPW
        ins = ((h_hbm, hi_v, hp_v, ent_hbm, hr_v),
               (t_hbm, ti_v, tp_v, ent_hbm, tr_v),
               (r_hbm, ri_v, rp_v, rel_hbm, rr_v),
               (nh_hbm, nhi_v, nhp_v, ent_hbm, nhr_v),
               (nt_hbm, nti_v, ntp_v, ent_hbm, ntr_v))
        iota = lax.iota(jnp.int32, _G)
        for c in range(_NCH):
            off = base + c * _CH
            for idx_hbm, idx_v, par_v, _, _ in ins:
                pltpu.sync_copy(idx_hbm.at[pl.ds(off, _CH)], idx_v)
            for _, idx_v, par_v, _, _ in ins:
                def halve(j, idx_v=idx_v, par_v=par_v):
                    sl = pl.ds(j * _G, _G)
                    e = idx_v[sl]
                    par_v[sl] = ((e >> 11) & 1) << 6
                    idx_v[sl] = ((e >> 12) << 11) + (e & 2047)
                pl.loop(0, _CH // _G)(halve)
            copies = [pltpu.async_copy(tab_hbm.at[idx_v], rows_v, sem)
                      for _, idx_v, _, tab_hbm, rows_v in ins]
            for cp in copies:
                cp.wait()

            @pl.loop(0, _NG)
            def _(g):
                row0 = g * _G
                sl16 = pl.ds(row0, _G)
                ph16 = hp_v[sl16]
                pt16 = tp_v[sl16]
                pr16 = rp_v[sl16]
                pnh16 = nhp_v[sl16]
                pnt16 = ntp_v[sl16]
                for i in range(_G):
                    row = row0 + i
                    ph, pt, pr = ph16[i], pt16[i], pr16[i]
                    pnh, pnt = pnh16[i], pnt16[i]
                    p = jnp.zeros((_G,), jnp.float32)
                    pn = jnp.zeros((_G,), jnp.float32)
                    for kk in range(_EMB // _G):
                        o = kk * _G
                        rv = rr_v[row, pl.ds(pr + o, _G)]
                        d = hr_v[row, pl.ds(ph + o, _G)] + rv \
                            - tr_v[row, pl.ds(pt + o, _G)]
                        p = p + d * d
                        dn = nhr_v[row, pl.ds(pnh + o, _G)] + rv \
                            - ntr_v[row, pl.ds(pnt + o, _G)]
                        pn = pn + dn * dn
                    sp_v[i, :] = p
                    sn_v[i, :] = pn
                accp = jnp.zeros((_G,), jnp.float32)
                accn = jnp.zeros((_G,), jnp.float32)
                for j in range(_G):
                    col = jnp.full((_G,), j, jnp.int32)
                    accp = accp + plsc.load_gather(sp_v, [iota, col])
                    accn = accn + plsc.load_gather(sn_v, [iota, col])
                pos_v[pl.ds(row0, _G)] = _neg_sqrt(accp)
                neg_v[pl.ds(row0, _G)] = _neg_sqrt(accn)

            pltpu.sync_copy(pos_v, pos_hbm.at[pl.ds(off, _CH)])
            pltpu.sync_copy(neg_v, neg_hbm.at[pl.ds(off, _CH)])

    return k(ent2, rel2, heads, tails, relations, neg_heads, neg_tails)


_TW = 4096


def _tc_pack(tab_t):
    n = tab_t.shape[1]
    nb = (n + _TW - 1) // _TW

    def body(x_ref, o_ref):
        y = jnp.transpose(x_ref[...])
        o_ref[...] = jnp.concatenate(
            [y[: _TW // 2], y[_TW // 2:]], axis=1)

    return pl.pallas_call(
        body,
        grid=(nb,),
        in_specs=[pl.BlockSpec((_EMB, _TW), lambda i: (0, i))],
        out_specs=pl.BlockSpec((_TW // 2, 2 * _EMB), lambda i: (i, 0)),
        out_shape=jax.ShapeDtypeStruct((nb * _TW // 2, 2 * _EMB), jnp.float32),
        compiler_params=pltpu.CompilerParams(
            dimension_semantics=("parallel",)),
    )(tab_t)


def kernel(heads, tails, relations, negative_heads, negative_tails, ent_emb, rel_emb):
    idx = [x.astype(jnp.int32) for x in
           (heads, tails, relations, negative_heads, negative_tails)]
    ent2 = _tc_pack(jnp.swapaxes(ent_emb, 0, 1))
    rel2 = _tc_pack(jnp.swapaxes(rel_emb, 0, 1))
    pos, neg = _sc_score(ent2, rel2, *idx)
    return (pos, neg)
